# Initial kernel scaffold; baseline (speedup 1.0000x reference)
import jax
import jax.numpy as jnp
from jax.experimental import pallas as pl


def kernel(x, batch, v):
    raise NotImplementedError("write your pallas kernel here")


# speedup vs baseline: 94.2494x; 94.2494x over previous
import functools

import jax
import jax.numpy as jnp
import numpy as np
from jax.experimental import pallas as pl

N = 32768
NUM_THETAS = 32
BUMP_STEPS = 32
R = 1.1
NUM_SEGMENTS = 16
BLK = 2048


def _ect_kernel(x_ref, batch_ref, linp_ref, v0_ref, v1_ref, out_ref):
    i = pl.program_id(0)
    xb = x_ref[...]
    x0 = xb[:, 0:1]
    x1 = xb[:, 1:2]
    z = x0 * v0_ref[...] + x1 * v1_ref[...]
    sig = jax.nn.sigmoid(linp_ref[...] - z)
    seg = batch_ref[...]
    row = jax.lax.broadcasted_iota(jnp.int32, (NUM_SEGMENTS, BLK), 0)
    oh = (seg == row).astype(jnp.float32)
    acc = jnp.dot(oh, sig, preferred_element_type=jnp.float32)

    @pl.when(i == 0)
    def _():
        out_ref[...] = jnp.zeros_like(out_ref)

    out_ref[...] += acc


@jax.jit
def kernel(x, batch, v):
    st = BUMP_STEPS * NUM_THETAS
    lin = jnp.linspace(-R, R, BUMP_STEPS, dtype=jnp.float32)
    linp = jnp.repeat(200.0 * lin, NUM_THETAS).reshape(1, st)
    v0 = jnp.tile(200.0 * v[:, 0], BUMP_STEPS).reshape(1, st)
    v1 = jnp.tile(200.0 * v[:, 1], BUMP_STEPS).reshape(1, st)
    batch2 = batch.reshape(1, N)

    out = pl.pallas_call(
        _ect_kernel,
        grid=(N // BLK,),
        in_specs=[
            pl.BlockSpec((BLK, 2), lambda i: (i, 0)),
            pl.BlockSpec((1, BLK), lambda i: (0, i)),
            pl.BlockSpec((1, st), lambda i: (0, 0)),
            pl.BlockSpec((1, st), lambda i: (0, 0)),
            pl.BlockSpec((1, st), lambda i: (0, 0)),
        ],
        out_specs=pl.BlockSpec((NUM_SEGMENTS, st), lambda i: (0, 0)),
        out_shape=jax.ShapeDtypeStruct((NUM_SEGMENTS, st), jnp.float32),
    )(x, batch2, linp, v0, v1)
    return out.reshape(NUM_SEGMENTS, BUMP_STEPS, NUM_THETAS)


# speedup vs baseline: 134.8628x; 1.4309x over previous
import jax
import jax.numpy as jnp
import numpy as np
from jax.experimental import pallas as pl

N = 32768
NUM_THETAS = 32
BUMP_STEPS = 32
R = 1.1
NUM_SEGMENTS = 16
BLK = 2048


def _ect_kernel(a_ref, batch_ref, w_ref, out_ref):
    i = pl.program_id(0)
    y = jnp.dot(a_ref[...], w_ref[...], preferred_element_type=jnp.float32)
    th = jnp.tanh(y).astype(jnp.bfloat16)
    seg = batch_ref[...]
    row = jax.lax.broadcasted_iota(jnp.int32, (NUM_SEGMENTS, BLK), 0)
    ohb = (seg == row).astype(jnp.bfloat16)
    acc = jnp.dot(ohb, th, preferred_element_type=jnp.float32)
    cnt = jnp.sum(ohb.astype(jnp.float32), axis=1, keepdims=True)

    @pl.when(i == 0)
    def _():
        out_ref[...] = jnp.zeros_like(out_ref)

    out_ref[...] += 0.5 * acc + 0.5 * cnt


@jax.jit
def kernel(x, batch, v):
    st = BUMP_STEPS * NUM_THETAS
    lin = jnp.linspace(-R, R, BUMP_STEPS, dtype=jnp.float32)
    linp = jnp.repeat(100.0 * lin, NUM_THETAS).reshape(1, st)
    v0 = jnp.tile(-100.0 * v[:, 0], BUMP_STEPS).reshape(1, st)
    v1 = jnp.tile(-100.0 * v[:, 1], BUMP_STEPS).reshape(1, st)
    wf = jnp.concatenate([v0, v1, linp], axis=0)
    w_hi = wf.astype(jnp.bfloat16)
    w_lo = (wf - w_hi.astype(jnp.float32)).astype(jnp.bfloat16)
    w = jnp.concatenate([w_hi, w_lo], axis=0)
    xb = x.astype(jnp.bfloat16)
    ones = jnp.ones((N, 1), jnp.bfloat16)
    a = jnp.concatenate([xb, ones, xb, ones], axis=1)
    batch2 = batch.reshape(1, N)

    out = pl.pallas_call(
        _ect_kernel,
        grid=(N // BLK,),
        in_specs=[
            pl.BlockSpec((BLK, 6), lambda i: (i, 0)),
            pl.BlockSpec((1, BLK), lambda i: (0, i)),
            pl.BlockSpec((6, st), lambda i: (0, 0)),
        ],
        out_specs=pl.BlockSpec((NUM_SEGMENTS, st), lambda i: (0, 0)),
        out_shape=jax.ShapeDtypeStruct((NUM_SEGMENTS, st), jnp.float32),
    )(a, batch2, w)
    return out.reshape(NUM_SEGMENTS, BUMP_STEPS, NUM_THETAS)


# speedup vs baseline: 151.6460x; 1.1244x over previous
import jax
import jax.numpy as jnp
import numpy as np
from jax.experimental import pallas as pl
from jax.experimental.pallas import tpu as pltpu

N = 32768
NUM_THETAS = 32
BUMP_STEPS = 32
R = 1.1
NUM_SEGMENTS = 16
BLK = 4096

_LIN = np.linspace(-R, R, BUMP_STEPS, dtype=np.float32)
_LINP = np.repeat(100.0 * _LIN, NUM_THETAS).reshape(1, BUMP_STEPS * NUM_THETAS)


def _ect_kernel(x_ref, batch_ref, v_ref, linp_ref, out_ref, acc_ref):
    i = pl.program_id(0)

    @pl.when(i == 0)
    def _():
        acc_ref[...] = jnp.zeros_like(acc_ref)

    xs = x_ref[...] * (-100.0)
    nhn = jax.lax.dot_general(
        xs, v_ref[...], (((1,), (1,)), ((), ())),
        preferred_element_type=jnp.float32)
    y = jnp.tile(nhn, (1, BUMP_STEPS)) + linp_ref[...]
    th = jnp.tanh(y)
    seg = batch_ref[...]
    row = jax.lax.broadcasted_iota(jnp.int32, (NUM_SEGMENTS, BLK), 0)
    oh = (seg == row).astype(jnp.float32)
    cnt = jnp.sum(oh, axis=1, keepdims=True)
    acc = jnp.dot(oh, th, preferred_element_type=jnp.float32)
    acc_ref[...] += 0.5 * acc + 0.5 * cnt

    @pl.when(i == (N // BLK) - 1)
    def _():
        out_ref[...] = acc_ref[...].reshape(
            NUM_SEGMENTS, BUMP_STEPS, NUM_THETAS)


@jax.jit
def kernel(x, batch, v):
    st = BUMP_STEPS * NUM_THETAS
    return pl.pallas_call(
        _ect_kernel,
        grid=(N // BLK,),
        in_specs=[
            pl.BlockSpec((BLK, 2), lambda i: (i, 0)),
            pl.BlockSpec((1, BLK), lambda i: (0, i)),
            pl.BlockSpec((NUM_THETAS, 2), lambda i: (0, 0)),
            pl.BlockSpec((1, st), lambda i: (0, 0)),
        ],
        out_specs=pl.BlockSpec(
            (NUM_SEGMENTS, BUMP_STEPS, NUM_THETAS), lambda i: (0, 0, 0)),
        out_shape=jax.ShapeDtypeStruct(
            (NUM_SEGMENTS, BUMP_STEPS, NUM_THETAS), jnp.float32),
        scratch_shapes=[pltpu.VMEM((NUM_SEGMENTS, st), jnp.float32)],
    )(x, batch.reshape(1, N), v, jnp.asarray(_LINP))


# speedup vs baseline: 152.4440x; 1.0053x over previous
import jax
import jax.numpy as jnp
import numpy as np
from jax.experimental import pallas as pl
from jax.experimental.pallas import tpu as pltpu

N = 32768
NUM_THETAS = 32
BUMP_STEPS = 32
R = 1.1
NUM_SEGMENTS = 16
BLK = 8192

_LIN = np.linspace(-R, R, BUMP_STEPS, dtype=np.float32)
_LINP = np.repeat(100.0 * _LIN, NUM_THETAS).reshape(1, BUMP_STEPS * NUM_THETAS)


def _ect_kernel(x_ref, batch_ref, v_ref, linp_ref, out_ref, acc_ref):
    i = pl.program_id(0)

    @pl.when(i == 0)
    def _():
        acc_ref[...] = jnp.zeros_like(acc_ref)

    xs = x_ref[...] * (-100.0)
    nhn = jax.lax.dot_general(
        xs, v_ref[...], (((1,), (1,)), ((), ())),
        preferred_element_type=jnp.float32)
    y = jnp.tile(nhn, (1, BUMP_STEPS)) + linp_ref[...]
    th = jnp.tanh(y)
    seg = batch_ref[...]
    row = jax.lax.broadcasted_iota(jnp.int32, (NUM_SEGMENTS, BLK), 0)
    oh = (seg == row).astype(jnp.float32)
    cnt = jnp.sum(oh, axis=1, keepdims=True)
    acc = jnp.dot(oh, th, preferred_element_type=jnp.float32)
    acc_ref[...] += 0.5 * acc + 0.5 * cnt

    @pl.when(i == (N // BLK) - 1)
    def _():
        out_ref[...] = acc_ref[...].reshape(
            NUM_SEGMENTS, BUMP_STEPS, NUM_THETAS)


@jax.jit
def kernel(x, batch, v):
    st = BUMP_STEPS * NUM_THETAS
    return pl.pallas_call(
        _ect_kernel,
        grid=(N // BLK,),
        in_specs=[
            pl.BlockSpec((BLK, 2), lambda i: (i, 0)),
            pl.BlockSpec((1, BLK), lambda i: (0, i)),
            pl.BlockSpec((NUM_THETAS, 2), lambda i: (0, 0)),
            pl.BlockSpec((1, st), lambda i: (0, 0)),
        ],
        out_specs=pl.BlockSpec(
            (NUM_SEGMENTS, BUMP_STEPS, NUM_THETAS), lambda i: (0, 0, 0)),
        out_shape=jax.ShapeDtypeStruct(
            (NUM_SEGMENTS, BUMP_STEPS, NUM_THETAS), jnp.float32),
        scratch_shapes=[pltpu.VMEM((NUM_SEGMENTS, st), jnp.float32)],
    )(x, batch.reshape(1, N), v, jnp.asarray(_LINP))


# speedup vs baseline: 159.9763x; 1.0494x over previous
import jax
import jax.numpy as jnp
import numpy as np
from jax.experimental import pallas as pl
from jax.experimental.pallas import tpu as pltpu

N = 32768
NUM_THETAS = 32
BUMP_STEPS = 32
R = 1.1
NUM_SEGMENTS = 16
BLK = 8192
KCHUNK = 2048

_LIN = np.linspace(-R, R, BUMP_STEPS, dtype=np.float32)
_LINP = np.repeat(100.0 * _LIN, NUM_THETAS).reshape(1, BUMP_STEPS * NUM_THETAS)


def _ect_kernel(x_ref, batch_ref, v_ref, linp_ref, out_ref, acc_ref):
    i = pl.program_id(0)

    @pl.when(i == 0)
    def _():
        acc_ref[...] = jnp.zeros_like(acc_ref)

    xs = x_ref[...] * (-100.0)
    nhn = jax.lax.dot_general(
        xs, v_ref[...], (((1,), (1,)), ((), ())),
        preferred_element_type=jnp.float32)

    seg = batch_ref[...]
    row = jax.lax.broadcasted_iota(jnp.int32, (NUM_SEGMENTS, BLK), 0)
    oh = (seg == row).astype(jnp.float32)
    cnt = jnp.sum(oh, axis=1, keepdims=True)
    ohb = oh.astype(jnp.bfloat16)

    acc = None
    for k in range(BLK // KCHUNK):
        sl = slice(k * KCHUNK, (k + 1) * KCHUNK)
        y = jnp.tile(nhn[sl, :], (1, BUMP_STEPS)) + linp_ref[...]
        th = jnp.tanh(y).astype(jnp.bfloat16)
        d = jnp.dot(ohb[:, sl], th, preferred_element_type=jnp.float32)
        acc = d if acc is None else acc + d
    acc_ref[...] += 0.5 * acc + 0.5 * cnt

    @pl.when(i == (N // BLK) - 1)
    def _():
        out_ref[...] = acc_ref[...].reshape(
            NUM_SEGMENTS, BUMP_STEPS, NUM_THETAS)


@jax.jit
def kernel(x, batch, v):
    st = BUMP_STEPS * NUM_THETAS
    return pl.pallas_call(
        _ect_kernel,
        grid=(N // BLK,),
        in_specs=[
            pl.BlockSpec((BLK, 2), lambda i: (i, 0)),
            pl.BlockSpec((1, BLK), lambda i: (0, i)),
            pl.BlockSpec((NUM_THETAS, 2), lambda i: (0, 0)),
            pl.BlockSpec((1, st), lambda i: (0, 0)),
        ],
        out_specs=pl.BlockSpec(
            (NUM_SEGMENTS, BUMP_STEPS, NUM_THETAS), lambda i: (0, 0, 0)),
        out_shape=jax.ShapeDtypeStruct(
            (NUM_SEGMENTS, BUMP_STEPS, NUM_THETAS), jnp.float32),
        scratch_shapes=[pltpu.VMEM((NUM_SEGMENTS, st), jnp.float32)],
    )(x, batch.reshape(1, N), v, jnp.asarray(_LINP))
